# baseline (speedup 1.0000x reference)
import jax
import jax.numpy as jnp
from jax import lax
from jax.experimental import pallas as pl
from jax.experimental.pallas import tpu as pltpu
from jax.experimental.pallas import tpu_sc as plsc

N = 10000
D = 128
E = 320000
EPS = 1e-12

NPAD = 10112
NS = 16
B = 128
NB = 160
K = 32
RING = 2
DRING = 4
EPAD = NS * NB * B
ROWS_PER_TILE = NPAD // NS
MM_BLK = NPAD // 16
_PREC = lax.Precision.HIGHEST


def _mm3_body(h_ref, w1_ref, w2_ref, wr_ref, br_ref, o1_ref, o2_ref, or_ref):
    h = h_ref[...]
    o1_ref[...] = jnp.dot(h, w1_ref[...], precision=_PREC,
                          preferred_element_type=jnp.float32)
    o2_ref[...] = jnp.dot(h, w2_ref[...], precision=_PREC,
                          preferred_element_type=jnp.float32)
    or_ref[...] = (
        jnp.dot(h, wr_ref[...], precision=_PREC,
                preferred_element_type=jnp.float32) + br_ref[...]
    )


def _mm3(h, w1, w2, wr, br):
    grid = (NPAD // MM_BLK,)
    return pl.pallas_call(
        _mm3_body,
        grid=grid,
        in_specs=[
            pl.BlockSpec((MM_BLK, D), lambda i: (i, 0)),
            pl.BlockSpec((D, D), lambda i: (0, 0)),
            pl.BlockSpec((D, D), lambda i: (0, 0)),
            pl.BlockSpec((D, D), lambda i: (0, 0)),
            pl.BlockSpec((1, D), lambda i: (0, 0)),
        ],
        out_specs=[
            pl.BlockSpec((MM_BLK, D), lambda i: (i, 0)),
            pl.BlockSpec((MM_BLK, D), lambda i: (i, 0)),
            pl.BlockSpec((MM_BLK, D), lambda i: (i, 0)),
        ],
        out_shape=[jax.ShapeDtypeStruct((NPAD, D), jnp.float32)] * 3,
    )(h, w1, w2, wr, br.reshape(1, D))


def _combine_body(hr_ref, a1_ref, a2_ref, d1_ref, d2_ref, o_ref):
    i = pl.program_id(0)
    row = lax.broadcasted_iota(jnp.int32, (MM_BLK, 1), 0) + i * MM_BLK
    deg1 = jnp.maximum(d1_ref[:, 0:1], 1.0)
    deg2 = jnp.maximum(d2_ref[:, 0:1], 1.0)
    v = hr_ref[...] + a1_ref[...] / deg1 + a2_ref[...] / deg2
    v = jnp.where(row < N, v, 0.0)
    nrm = jnp.sqrt(jnp.sum(v * v, axis=1, keepdims=True))
    o_ref[...] = v / jnp.maximum(nrm, EPS)


def _combine(hr, a1, a2, d1, d2):
    grid = (NPAD // MM_BLK,)
    return pl.pallas_call(
        _combine_body,
        grid=grid,
        in_specs=[pl.BlockSpec((MM_BLK, D), lambda i: (i, 0))] * 5,
        out_specs=pl.BlockSpec((MM_BLK, D), lambda i: (i, 0)),
        out_shape=jax.ShapeDtypeStruct((NPAD, D), jnp.float32),
    )(hr, a1, a2, d1, d2)


def _fill_vmem_rows(buf, nrows, ncols, value):
    v = jnp.full((16,), value, jnp.float32)

    def body(i, _):
        for j in range(ncols // 16):
            buf[i, pl.ds(j * 16, 16)] = v
        return 0

    lax.fori_loop(0, nrows, body, 0)


_SC_MESH = plsc.VectorSubcoreMesh(core_axis_name="c", subcore_axis_name="s")
_SC_OUT2 = [jax.ShapeDtypeStruct((NPAD, D), jnp.float32)] * 2


def _zero_acc_slice(s, rows_v, acc):
    base = s * ROWS_PER_TILE
    nfull = ROWS_PER_TILE // B
    for k in range(nfull):
        pltpu.sync_copy(rows_v, acc.at[pl.ds(base + k * B, B)])
    rem = ROWS_PER_TILE - nfull * B
    if rem:
        pltpu.sync_copy(
            rows_v.at[pl.ds(0, rem)], acc.at[pl.ds(base + nfull * B, rem)]
        )


def _dump_acc_slice(s, acc, out_hbm):
    pltpu.sync_copy(
        acc.at[pl.ds(s * ROWS_PER_TILE, ROWS_PER_TILE)],
        out_hbm.at[pl.ds(s * ROWS_PER_TILE, ROWS_PER_TILE)],
    )


def _sc_pass_body(h1, h2, src_hbm, dst_hbm, agg1, agg2,
                  ga_v, gb_v, sa_v, sb_v, b0, b1, acc, g0, g1):
    c = lax.axis_index("c")
    s = lax.axis_index("s")

    def direction(tbl, gidx_hbm, sidx_hbm, out_hbm):
        _fill_vmem_rows(b0, B, D, 0.0)
        _zero_acc_slice(s, b0, acc)
        plsc.subcore_barrier()

        def body(i, _):
            base = (s * NB + i) * B
            pltpu.sync_copy(gidx_hbm.at[pl.ds(base, B)], ga_v)
            pltpu.sync_copy(sidx_hbm.at[pl.ds(base, B)], sa_v)
            pltpu.async_copy(tbl.at[ga_v], b0, g0).wait()
            pltpu.sync_copy(b0, acc.at[sa_v], add=True)
            return 0

        lax.fori_loop(0, NB, body, 0)
        plsc.subcore_barrier()
        _dump_acc_slice(s, acc, out_hbm)

    @pl.when(c == 0)
    def _():
        direction(h1, src_hbm, dst_hbm, agg1)

    @pl.when(c == 1)
    def _():
        direction(h2, dst_hbm, src_hbm, agg2)


_sc_pass = pl.kernel(
    _sc_pass_body,
    out_type=_SC_OUT2,
    mesh=_SC_MESH,
    scratch_types=[
        pltpu.VMEM((B,), jnp.int32),
        pltpu.VMEM((B,), jnp.int32),
        pltpu.VMEM((B,), jnp.int32),
        pltpu.VMEM((B,), jnp.int32),
        pltpu.VMEM((B, D), jnp.float32),
        pltpu.VMEM((B, D), jnp.float32),
        pltpu.VMEM_SHARED((NPAD, D), jnp.float32),
    ] + [pltpu.SemaphoreType.DMA] * 2,
)


def _sc_deg_body(src_hbm, dst_hbm, deg1, deg2,
                 sidx_v, ones_v, acc, s0, s1, s2, s3):
    c = lax.axis_index("c")
    s = lax.axis_index("s")
    sems = (s0, s1, s2, s3)

    def direction(sidx_hbm, out_hbm):
        _fill_vmem_rows(ones_v, B, D, 0.0)
        _zero_acc_slice(s, ones_v, acc)
        _fill_vmem_rows(ones_v, B, D, 1.0)
        plsc.subcore_barrier()

        def chunk(ch, _):
            rowbase = s * NB + ch * K
            pltpu.sync_copy(sidx_hbm.at[pl.ds(rowbase, K)], sidx_v)

            def quad(t, _):
                base = DRING * t
                for k in range(DRING):
                    @pl.when(t > 0)
                    def _():
                        pltpu.make_async_copy(
                            ones_v, acc.at[sidx_v.at[base - DRING + k]], sems[k]
                        ).wait()

                    pltpu.make_async_copy(
                        ones_v, acc.at[sidx_v.at[base + k]], sems[k]
                    ).start(add=True)
                return 0

            lax.fori_loop(0, K // DRING, quad, 0)
            for k in range(DRING):
                pltpu.make_async_copy(
                    ones_v, acc.at[sidx_v.at[K - DRING + k]], sems[k]
                ).wait()
            return 0

        lax.fori_loop(0, NB // K, chunk, 0)
        plsc.subcore_barrier()
        _dump_acc_slice(s, acc, out_hbm)

    @pl.when(c == 0)
    def _():
        direction(dst_hbm, deg1)

    @pl.when(c == 1)
    def _():
        direction(src_hbm, deg2)


_sc_deg = pl.kernel(
    _sc_deg_body,
    out_type=_SC_OUT2,
    mesh=_SC_MESH,
    scratch_types=[
        pltpu.VMEM((K, B), jnp.int32),
        pltpu.VMEM((B, D), jnp.float32),
        pltpu.VMEM_SHARED((NPAD, D), jnp.float32),
    ] + [pltpu.SemaphoreType.DMA] * 4,
)


def kernel(x, edge_index, W1_0, W2_0, Wr_0, br_0, W1_1, W2_1, Wr_1, br_1):
    xp = jnp.zeros((NPAD, D), jnp.float32).at[:N].set(x)
    srcp = jnp.full((EPAD,), N, jnp.int32).at[:E].set(edge_index[0])
    dstp = jnp.full((EPAD,), N, jnp.int32).at[:E].set(edge_index[1])
    src2d = srcp.reshape(NS * NB, B)
    dst2d = dstp.reshape(NS * NB, B)

    deg1, deg2 = _sc_deg(src2d, dst2d)

    h1, h2, hr = _mm3(xp, W1_0, W2_0, Wr_0, br_0)
    a1, a2 = _sc_pass(h1, h2, srcp, dstp)
    y1 = _combine(hr, a1, a2, deg1, deg2)

    h1b, h2b, hrb = _mm3(y1, W1_1, W2_1, Wr_1, br_1)
    a1b, a2b = _sc_pass(h1b, h2b, srcp, dstp)
    y2 = _combine(hrb, a1b, a2b, deg1, deg2)

    return (x, y1[:N], y2[:N])


# speedup vs baseline: 1.4876x; 1.4876x over previous
import jax
import jax.numpy as jnp
from jax import lax
from jax.experimental import pallas as pl
from jax.experimental.pallas import tpu as pltpu
from jax.experimental.pallas import tpu_sc as plsc

N = 10000
D = 128
E = 320000
EPS = 1e-12

NPAD = 10240
NS = 16
B = 128
NB = -(-E // (NS * B))
EPAD = NS * NB * B
ROWS_PER_TILE = NPAD // NS
MM_BLK = 1024
_PREC = lax.Precision.HIGHEST


def _mm3_body(h_ref, w1_ref, w2_ref, wr_ref, br_ref, o1_ref, o2_ref, or_ref):
    h = h_ref[...]
    o1_ref[...] = jnp.dot(h, w1_ref[...], precision=_PREC,
                          preferred_element_type=jnp.float32)
    o2_ref[...] = jnp.dot(h, w2_ref[...], precision=_PREC,
                          preferred_element_type=jnp.float32)
    or_ref[...] = (
        jnp.dot(h, wr_ref[...], precision=_PREC,
                preferred_element_type=jnp.float32) + br_ref[...]
    )


def _mm3(h, w1, w2, wr, br):
    grid = (NPAD // MM_BLK,)
    return pl.pallas_call(
        _mm3_body,
        grid=grid,
        in_specs=[
            pl.BlockSpec((MM_BLK, D), lambda i: (i, 0)),
            pl.BlockSpec((D, D), lambda i: (0, 0)),
            pl.BlockSpec((D, D), lambda i: (0, 0)),
            pl.BlockSpec((D, D), lambda i: (0, 0)),
            pl.BlockSpec((1, D), lambda i: (0, 0)),
        ],
        out_specs=[
            pl.BlockSpec((MM_BLK, D), lambda i: (i, 0)),
            pl.BlockSpec((MM_BLK, D), lambda i: (i, 0)),
            pl.BlockSpec((MM_BLK, D), lambda i: (i, 0)),
        ],
        out_shape=[jax.ShapeDtypeStruct((NPAD, D), jnp.float32)] * 3,
    )(h, w1, w2, wr, br.reshape(1, D))


def _combine_body(hr_ref, a1_ref, a2_ref, d1_ref, d2_ref, o_ref):
    i = pl.program_id(0)
    row = lax.broadcasted_iota(jnp.int32, (MM_BLK, 1), 0) + i * MM_BLK
    deg1 = jnp.maximum(d1_ref[:, 0:1], 1.0)
    deg2 = jnp.maximum(d2_ref[:, 0:1], 1.0)
    v = hr_ref[...] + a1_ref[...] / deg1 + a2_ref[...] / deg2
    v = jnp.where(row < N, v, 0.0)
    nrm = jnp.sqrt(jnp.sum(v * v, axis=1, keepdims=True))
    o_ref[...] = v / jnp.maximum(nrm, EPS)


def _combine(hr, a1, a2, d1, d2):
    grid = (NPAD // MM_BLK,)
    return pl.pallas_call(
        _combine_body,
        grid=grid,
        in_specs=[pl.BlockSpec((MM_BLK, D), lambda i: (i, 0))] * 5,
        out_specs=pl.BlockSpec((MM_BLK, D), lambda i: (i, 0)),
        out_shape=jax.ShapeDtypeStruct((NPAD, D), jnp.float32),
    )(hr, a1, a2, d1, d2)


def _fill_vmem_rows(buf, nrows, ncols, value):
    v = jnp.full((16,), value, jnp.float32)

    def body(i, _):
        for j in range(ncols // 16):
            buf[i, pl.ds(j * 16, 16)] = v
        return 0

    lax.fori_loop(0, nrows, body, 0)


_SC_MESH = plsc.VectorSubcoreMesh(core_axis_name="c", subcore_axis_name="s")
_SC_OUT2 = [jax.ShapeDtypeStruct((NPAD, D), jnp.float32)] * 2


def _zero_acc_slice(s, rows_v, acc):
    for k in range(ROWS_PER_TILE // B):
        pltpu.sync_copy(rows_v, acc.at[pl.ds(s * ROWS_PER_TILE + k * B, B)])


def _dump_acc_slice(s, acc, out_hbm):
    pltpu.sync_copy(
        acc.at[pl.ds(s * ROWS_PER_TILE, ROWS_PER_TILE)],
        out_hbm.at[pl.ds(s * ROWS_PER_TILE, ROWS_PER_TILE)],
    )


def _sc_pass_body(h1, h2, src_hbm, dst_hbm, agg1, agg2,
                  gidx_v, sidx_v, rows_v, acc, sem):
    c = lax.axis_index("c")
    s = lax.axis_index("s")

    def direction(tbl, gidx_hbm, sidx_hbm, out_hbm):
        _fill_vmem_rows(rows_v, B, D, 0.0)
        _zero_acc_slice(s, rows_v, acc)
        plsc.subcore_barrier()

        def body(i, _):
            base = (s * NB + i) * B
            pltpu.sync_copy(gidx_hbm.at[pl.ds(base, B)], gidx_v)
            pltpu.sync_copy(sidx_hbm.at[pl.ds(base, B)], sidx_v)
            pltpu.async_copy(tbl.at[gidx_v], rows_v, sem).wait()
            pltpu.sync_copy(rows_v, acc.at[sidx_v], add=True)
            return 0

        lax.fori_loop(0, NB, body, 0)
        plsc.subcore_barrier()
        _dump_acc_slice(s, acc, out_hbm)

    @pl.when(c == 0)
    def _():
        direction(h1, src_hbm, dst_hbm, agg1)

    @pl.when(c == 1)
    def _():
        direction(h2, dst_hbm, src_hbm, agg2)


_sc_pass = pl.kernel(
    _sc_pass_body,
    out_type=_SC_OUT2,
    mesh=_SC_MESH,
    scratch_types=[
        pltpu.VMEM((B,), jnp.int32),
        pltpu.VMEM((B,), jnp.int32),
        pltpu.VMEM((B, D), jnp.float32),
        pltpu.VMEM_SHARED((NPAD, D), jnp.float32),
        pltpu.SemaphoreType.DMA,
    ],
)


def _sc_deg_body(src_hbm, dst_hbm, deg1, deg2, sidx_v, ones_v, acc, sem):
    c = lax.axis_index("c")
    s = lax.axis_index("s")

    def direction(sidx_hbm, out_hbm):
        _fill_vmem_rows(ones_v, B, D, 0.0)
        _zero_acc_slice(s, ones_v, acc)
        _fill_vmem_rows(ones_v, B, D, 1.0)
        plsc.subcore_barrier()

        def body(i, _):
            base = (s * NB + i) * B
            pltpu.sync_copy(sidx_hbm.at[pl.ds(base, B)], sidx_v)
            pltpu.sync_copy(ones_v, acc.at[sidx_v], add=True)
            return 0

        lax.fori_loop(0, NB, body, 0)
        plsc.subcore_barrier()
        _dump_acc_slice(s, acc, out_hbm)

    @pl.when(c == 0)
    def _():
        direction(dst_hbm, deg1)

    @pl.when(c == 1)
    def _():
        direction(src_hbm, deg2)


_sc_deg = pl.kernel(
    _sc_deg_body,
    out_type=_SC_OUT2,
    mesh=_SC_MESH,
    scratch_types=[
        pltpu.VMEM((B,), jnp.int32),
        pltpu.VMEM((B, D), jnp.float32),
        pltpu.VMEM_SHARED((NPAD, D), jnp.float32),
        pltpu.SemaphoreType.DMA,
    ],
)


def kernel(x, edge_index, W1_0, W2_0, Wr_0, br_0, W1_1, W2_1, Wr_1, br_1):
    xp = jnp.zeros((NPAD, D), jnp.float32).at[:N].set(x)
    srcp = jnp.full((EPAD,), N, jnp.int32).at[:E].set(edge_index[0])
    dstp = jnp.full((EPAD,), N, jnp.int32).at[:E].set(edge_index[1])

    deg1, deg2 = _sc_deg(srcp, dstp)

    h1, h2, hr = _mm3(xp, W1_0, W2_0, Wr_0, br_0)
    a1, a2 = _sc_pass(h1, h2, srcp, dstp)
    y1 = _combine(hr, a1, a2, deg1, deg2)

    h1b, h2b, hrb = _mm3(y1, W1_1, W2_1, Wr_1, br_1)
    a1b, a2b = _sc_pass(h1b, h2b, srcp, dstp)
    y2 = _combine(hrb, a1b, a2b, deg1, deg2)

    return (x, y1[:N], y2[:N])


# speedup vs baseline: 1.6508x; 1.1096x over previous
import jax
import jax.numpy as jnp
from jax import lax
from jax.experimental import pallas as pl
from jax.experimental.pallas import tpu as pltpu
from jax.experimental.pallas import tpu_sc as plsc

N = 10000
D = 128
E = 320000
EPS = 1e-12

NPAD = 10240
NS = 16
B = 128
NB = -(-E // (NS * B))
EPAD = NS * NB * B
ROWS_PER_TILE = NPAD // NS
MM_BLK = 1024
_PREC = lax.Precision.HIGHEST


def _mm3_body(h_ref, w1_ref, w2_ref, wr_ref, br_ref, o1_ref, o2_ref, or_ref):
    h = h_ref[...]
    o1_ref[...] = jnp.dot(h, w1_ref[...], precision=_PREC,
                          preferred_element_type=jnp.float32)
    o2_ref[...] = jnp.dot(h, w2_ref[...], precision=_PREC,
                          preferred_element_type=jnp.float32)
    or_ref[...] = (
        jnp.dot(h, wr_ref[...], precision=_PREC,
                preferred_element_type=jnp.float32) + br_ref[...]
    )


def _mm3(h, w1, w2, wr, br):
    grid = (NPAD // MM_BLK,)
    return pl.pallas_call(
        _mm3_body,
        grid=grid,
        in_specs=[
            pl.BlockSpec((MM_BLK, D), lambda i: (i, 0)),
            pl.BlockSpec((D, D), lambda i: (0, 0)),
            pl.BlockSpec((D, D), lambda i: (0, 0)),
            pl.BlockSpec((D, D), lambda i: (0, 0)),
            pl.BlockSpec((1, D), lambda i: (0, 0)),
        ],
        out_specs=[
            pl.BlockSpec((MM_BLK, D), lambda i: (i, 0)),
            pl.BlockSpec((MM_BLK, D), lambda i: (i, 0)),
            pl.BlockSpec((MM_BLK, D), lambda i: (i, 0)),
        ],
        out_shape=[jax.ShapeDtypeStruct((NPAD, D), jnp.float32)] * 3,
    )(h, w1, w2, wr, br.reshape(1, D))


def _combine_body(hr_ref, a1_ref, a2_ref, d1_ref, d2_ref, o_ref):
    i = pl.program_id(0)
    row = lax.broadcasted_iota(jnp.int32, (MM_BLK, 1), 0) + i * MM_BLK
    deg1 = jnp.maximum(d1_ref[:, 0:1], 1.0)
    deg2 = jnp.maximum(d2_ref[:, 0:1], 1.0)
    v = hr_ref[...] + a1_ref[...] / deg1 + a2_ref[...] / deg2
    v = jnp.where(row < N, v, 0.0)
    nrm = jnp.sqrt(jnp.sum(v * v, axis=1, keepdims=True))
    o_ref[...] = v / jnp.maximum(nrm, EPS)


def _combine(hr, a1, a2, d1, d2):
    grid = (NPAD // MM_BLK,)
    return pl.pallas_call(
        _combine_body,
        grid=grid,
        in_specs=[pl.BlockSpec((MM_BLK, D), lambda i: (i, 0))] * 5,
        out_specs=pl.BlockSpec((MM_BLK, D), lambda i: (i, 0)),
        out_shape=jax.ShapeDtypeStruct((NPAD, D), jnp.float32),
    )(hr, a1, a2, d1, d2)


def _fill_vmem_rows(buf, nrows, ncols, value):
    v = jnp.full((16,), value, jnp.float32)

    def body(i, _):
        for j in range(ncols // 16):
            buf[i, pl.ds(j * 16, 16)] = v
        return 0

    lax.fori_loop(0, nrows, body, 0)


_SC_MESH = plsc.VectorSubcoreMesh(core_axis_name="c", subcore_axis_name="s")
_SC_OUT2 = [jax.ShapeDtypeStruct((NPAD, D), jnp.float32)] * 2


def _zero_acc_slice(s, rows_v, acc):
    for k in range(ROWS_PER_TILE // B):
        pltpu.sync_copy(rows_v, acc.at[pl.ds(s * ROWS_PER_TILE + k * B, B)])


def _dump_acc_slice(s, acc, out_hbm):
    pltpu.sync_copy(
        acc.at[pl.ds(s * ROWS_PER_TILE, ROWS_PER_TILE)],
        out_hbm.at[pl.ds(s * ROWS_PER_TILE, ROWS_PER_TILE)],
    )


def _sc_pass_body(h1, h2, src_hbm, dst_hbm, agg1, agg2,
                  gidx_v, sidx_v, rows_v, acc, sem):
    c = lax.axis_index("c")
    s = lax.axis_index("s")

    def direction(tbl, gidx_hbm, sidx_hbm, out_hbm):
        _fill_vmem_rows(rows_v, B, D, 0.0)
        _zero_acc_slice(s, rows_v, acc)
        plsc.subcore_barrier()

        def body(i, _):
            base = (s * NB + i) * B
            pltpu.sync_copy(gidx_hbm.at[pl.ds(base, B)], gidx_v)
            pltpu.sync_copy(sidx_hbm.at[pl.ds(base, B)], sidx_v)
            pltpu.async_copy(tbl.at[gidx_v], rows_v, sem).wait()
            pltpu.sync_copy(rows_v, acc.at[sidx_v], add=True)
            return 0

        lax.fori_loop(0, NB, body, 0)
        plsc.subcore_barrier()
        _dump_acc_slice(s, acc, out_hbm)

    @pl.when(c == 0)
    def _():
        direction(h1, src_hbm, dst_hbm, agg1)

    @pl.when(c == 1)
    def _():
        direction(h2, dst_hbm, src_hbm, agg2)


_sc_pass = pl.kernel(
    _sc_pass_body,
    out_type=_SC_OUT2,
    mesh=_SC_MESH,
    scratch_types=[
        pltpu.VMEM((B,), jnp.int32),
        pltpu.VMEM((B,), jnp.int32),
        pltpu.VMEM((B, D), jnp.float32),
        pltpu.VMEM_SHARED((NPAD, D), jnp.float32),
        pltpu.SemaphoreType.DMA,
    ],
)


NBD = 160
KD = 32
DRING = 4
EPADD = NS * NBD * B


def _sc_deg_body(src_hbm, dst_hbm, deg1, deg2,
                 sidx_v, ones_v, acc, s0, s1, s2, s3):
    c = lax.axis_index("c")
    s = lax.axis_index("s")
    sems = (s0, s1, s2, s3)

    def direction(sidx_hbm, out_hbm):
        _fill_vmem_rows(ones_v, B, D, 0.0)
        _zero_acc_slice(s, ones_v, acc)
        _fill_vmem_rows(ones_v, B, D, 1.0)
        plsc.subcore_barrier()

        def chunk(ch, _):
            rowbase = s * NBD + ch * KD
            pltpu.sync_copy(sidx_hbm.at[pl.ds(rowbase, KD)], sidx_v)

            def quad(t, _):
                base = DRING * t
                for k in range(DRING):
                    @pl.when(t > 0)
                    def _():
                        pltpu.make_async_copy(
                            ones_v, acc.at[sidx_v.at[base - DRING + k]], sems[k]
                        ).wait()

                    pltpu.make_async_copy(
                        ones_v, acc.at[sidx_v.at[base + k]], sems[k]
                    ).start(add=True)
                return 0

            lax.fori_loop(0, KD // DRING, quad, 0)
            for k in range(DRING):
                pltpu.make_async_copy(
                    ones_v, acc.at[sidx_v.at[KD - DRING + k]], sems[k]
                ).wait()
            return 0

        lax.fori_loop(0, NBD // KD, chunk, 0)
        plsc.subcore_barrier()
        _dump_acc_slice(s, acc, out_hbm)

    @pl.when(c == 0)
    def _():
        direction(dst_hbm, deg1)

    @pl.when(c == 1)
    def _():
        direction(src_hbm, deg2)


_sc_deg = pl.kernel(
    _sc_deg_body,
    out_type=_SC_OUT2,
    mesh=_SC_MESH,
    scratch_types=[
        pltpu.VMEM((KD, B), jnp.int32),
        pltpu.VMEM((B, D), jnp.float32),
        pltpu.VMEM_SHARED((NPAD, D), jnp.float32),
    ] + [pltpu.SemaphoreType.DMA] * 4,
)


def kernel(x, edge_index, W1_0, W2_0, Wr_0, br_0, W1_1, W2_1, Wr_1, br_1):
    xp = jnp.zeros((NPAD, D), jnp.float32).at[:N].set(x)
    srcp = jnp.full((EPAD,), N, jnp.int32).at[:E].set(edge_index[0])
    dstp = jnp.full((EPAD,), N, jnp.int32).at[:E].set(edge_index[1])

    srcd = jnp.full((EPADD,), N, jnp.int32).at[:E].set(edge_index[0])
    dstd = jnp.full((EPADD,), N, jnp.int32).at[:E].set(edge_index[1])
    deg1, deg2 = _sc_deg(
        srcd.reshape(NS * NBD, B), dstd.reshape(NS * NBD, B)
    )

    h1, h2, hr = _mm3(xp, W1_0, W2_0, Wr_0, br_0)
    a1, a2 = _sc_pass(h1, h2, srcp, dstp)
    y1 = _combine(hr, a1, a2, deg1, deg2)

    h1b, h2b, hrb = _mm3(y1, W1_1, W2_1, Wr_1, br_1)
    a1b, a2b = _sc_pass(h1b, h2b, srcp, dstp)
    y2 = _combine(hrb, a1b, a2b, deg1, deg2)

    return (x, y1[:N], y2[:N])
